# baseline (speedup 1.0000x reference)
import functools

import jax
import jax.numpy as jnp
from jax import lax
from jax.experimental import pallas as pl
from jax.experimental.pallas import tpu as pltpu
from jax.experimental.pallas import tpu_sc as plsc

N_NODES = 10000
N_EDGES = 320000
D_IN = 128
D_HID = 256
N_CLASSES = 64

NC = 2
NS = 16
N_PAD = 10240
BLK = 128
SUB = 8
EDGE_ROWS = 2500
EDGE_ROWS_PAD = 2560
BLKS_PER_TILE = EDGE_ROWS_PAD // (NC * NS)
NCHUNK = BLKS_PER_TILE // SUB
NODES_PER_TILE = N_PAD // NS


def _seg_sum_body(d, with_count, x_hbm, src_hbm, dst_hbm, *refs):
    if with_count:
        (out_hbm, cnt_hbm, src_idx, dst_idx, rows, ones, zcnt,
         acc, cnt_acc, gsem, csem) = refs
    else:
        (out_hbm, src_idx, dst_idx, rows, acc, gsem) = refs
        cnt_hbm = cnt_acc = ones = zcnt = csem = None

    cid = lax.axis_index("c")
    sid = lax.axis_index("s")
    wid = cid * NS + sid

    def zero_rows(i, _):
        for k in range(d // 16):
            rows[i, pl.ds(k * 16, 16)] = jnp.zeros((16,), jnp.float32)
        return _
    lax.fori_loop(0, BLK, zero_rows, None)
    if with_count:
        def fill_cnt(i, _):
            ones[pl.ds(i * 16, 16)] = jnp.full((16,), 1.0, jnp.float32)
            return _
        lax.fori_loop(0, BLK // 16, fill_cnt, None)
        def zero_zcnt(i, _):
            zcnt[pl.ds(i * 16, 16)] = jnp.zeros((16,), jnp.float32)
            return _
        lax.fori_loop(0, NODES_PER_TILE // 16, zero_zcnt, None)

    base_n = sid * NODES_PER_TILE
    for k in range(NODES_PER_TILE // BLK):
        pltpu.sync_copy(rows, acc.at[pl.ds(base_n + k * BLK, BLK)])
    if with_count:
        pltpu.sync_copy(zcnt, cnt_acc.at[pl.ds(base_n, NODES_PER_TILE)])
    plsc.subcore_barrier()

    base_c = wid * NCHUNK
    pltpu.sync_copy(src_hbm.at[pl.ds(base_c, NCHUNK)], src_idx)
    pltpu.sync_copy(dst_hbm.at[pl.ds(base_c, NCHUNK)], dst_idx)

    def edge_block(j, _):
        r = j // SUB
        s = j % SUB
        pltpu.async_copy(x_hbm.at[src_idx.at[r, s]], rows, gsem).wait()
        pltpu.sync_copy(rows, acc.at[dst_idx.at[r, s]], add=True)
        if with_count:
            pltpu.async_copy(ones, cnt_acc.at[dst_idx.at[r, s]],
                             csem, add=True)
        return _
    lax.fori_loop(0, BLKS_PER_TILE, edge_block, None)
    if with_count:
        def drain_cnt(j, _):
            pltpu.make_async_copy(ones, cnt_acc.at[dst_idx.at[0, 0]],
                                  csem).wait()
            return _
        lax.fori_loop(0, BLKS_PER_TILE, drain_cnt, None)
    plsc.subcore_barrier()

    pltpu.sync_copy(acc.at[pl.ds(base_n, NODES_PER_TILE)],
                    out_hbm.at[cid, pl.ds(base_n, NODES_PER_TILE)])
    if with_count:
        pltpu.sync_copy(cnt_acc.at[pl.ds(base_n, NODES_PER_TILE)],
                        cnt_hbm.at[cid, pl.ds(base_n, NODES_PER_TILE)])


def _make_seg_sum(d, with_count):
    mesh = plsc.VectorSubcoreMesh(core_axis_name="c", subcore_axis_name="s",
                                  num_cores=NC, num_subcores=NS)
    out_type = [jax.ShapeDtypeStruct((NC, N_PAD, d), jnp.float32)]
    scratch = [
        pltpu.VMEM((NCHUNK, SUB, BLK), jnp.int32),
        pltpu.VMEM((NCHUNK, SUB, BLK), jnp.int32),
        pltpu.VMEM((BLK, d), jnp.float32),
    ]
    if with_count:
        out_type.append(jax.ShapeDtypeStruct((NC, N_PAD), jnp.float32))
        scratch += [
            pltpu.VMEM((BLK,), jnp.float32),
            pltpu.VMEM((NODES_PER_TILE,), jnp.float32),
        ]
    scratch += [pltpu.VMEM_SHARED((N_PAD, d), jnp.float32)]
    if with_count:
        scratch += [pltpu.VMEM_SHARED((N_PAD,), jnp.float32)]
    scratch += [pltpu.SemaphoreType.DMA] * (2 if with_count else 1)
    return pl.kernel(functools.partial(_seg_sum_body, d, with_count),
                     out_type=out_type, mesh=mesh, scratch_types=scratch,
                     name=f"sage_seg_sum_d{d}")


def _layer1_tc(P_ref, cnt_ref, x_ref, Wl1_ref, bl1_ref, Wr1_ref,
               Wl2_ref, bl2_ref, Wr2_ref, p_ref, q_ref):
    c = (cnt_ref[0] + cnt_ref[1]).reshape(-1, 1)
    mean = (P_ref[0] + P_ref[1]) * (1.0 / jnp.maximum(c, 1.0))
    h = jnp.dot(mean, Wl1_ref[...], preferred_element_type=jnp.float32)
    h = h + jnp.dot(x_ref[...], Wr1_ref[...], preferred_element_type=jnp.float32)
    h = jnp.maximum(h + bl1_ref[...], 0.0)
    p = jnp.dot(h, Wl2_ref[...], preferred_element_type=jnp.float32)
    p_ref[...] = jnp.concatenate(
        [p, jnp.zeros_like(p)], axis=1)
    q_ref[...] = (jnp.dot(h, Wr2_ref[...], preferred_element_type=jnp.float32)
                  + bl2_ref[...])


def _layer2_tc(P2_ref, cnt_ref, q_ref, o_ref):
    c = (cnt_ref[0] + cnt_ref[1]).reshape(-1, 1)
    agg = (P2_ref[0] + P2_ref[1])[:, :N_CLASSES]
    z = agg * (1.0 / jnp.maximum(c, 1.0)) + q_ref[...]
    m = jnp.max(z, axis=1, keepdims=True)
    e = jnp.exp(z - m)
    s = jnp.sum(e, axis=1, keepdims=True)
    o_ref[...] = z - m - jnp.log(s)


_ROWS_B = 1024


def kernel(x, edge_index, batch, Wl1, bl1, Wr1, Wl2, bl2, Wr2):
    del batch
    src = edge_index[0].astype(jnp.int32)
    dst = edge_index[1].astype(jnp.int32)
    pad = EDGE_ROWS_PAD * BLK - N_EDGES
    trash = N_NODES + jnp.arange(pad, dtype=jnp.int32) % (N_PAD - N_NODES)
    src3d = jnp.pad(src, (0, pad)).reshape(EDGE_ROWS_PAD // SUB, SUB, BLK)
    dst3d = jnp.concatenate([dst, trash]).reshape(
        EDGE_ROWS_PAD // SUB, SUB, BLK)
    xp = jnp.pad(x, ((0, N_PAD - N_NODES), (0, 0)))

    P1, cnt = _make_seg_sum(D_IN, True)(xp, src3d, dst3d)

    grid = (N_PAD // _ROWS_B,)
    p, q = pl.pallas_call(
        _layer1_tc,
        grid=grid,
        in_specs=[
            pl.BlockSpec((NC, _ROWS_B, D_IN), lambda i: (0, i, 0)),
            pl.BlockSpec((NC, _ROWS_B), lambda i: (0, i)),
            pl.BlockSpec((_ROWS_B, D_IN), lambda i: (i, 0)),
            pl.BlockSpec((D_IN, D_HID), lambda i: (0, 0)),
            pl.BlockSpec((1, D_HID), lambda i: (0, 0)),
            pl.BlockSpec((D_IN, D_HID), lambda i: (0, 0)),
            pl.BlockSpec((D_HID, N_CLASSES), lambda i: (0, 0)),
            pl.BlockSpec((1, N_CLASSES), lambda i: (0, 0)),
            pl.BlockSpec((D_HID, N_CLASSES), lambda i: (0, 0)),
        ],
        out_specs=[
            pl.BlockSpec((_ROWS_B, 2 * N_CLASSES), lambda i: (i, 0)),
            pl.BlockSpec((_ROWS_B, N_CLASSES), lambda i: (i, 0)),
        ],
        out_shape=[
            jax.ShapeDtypeStruct((N_PAD, 2 * N_CLASSES), jnp.float32),
            jax.ShapeDtypeStruct((N_PAD, N_CLASSES), jnp.float32),
        ],
    )(P1, cnt, xp, Wl1, bl1.reshape(1, D_HID), Wr1,
      Wl2, bl2.reshape(1, N_CLASSES), Wr2)

    (P2,) = _make_seg_sum(2 * N_CLASSES, False)(p, src3d, dst3d)

    out = pl.pallas_call(
        _layer2_tc,
        grid=grid,
        in_specs=[
            pl.BlockSpec((NC, _ROWS_B, 2 * N_CLASSES), lambda i: (0, i, 0)),
            pl.BlockSpec((NC, _ROWS_B), lambda i: (0, i)),
            pl.BlockSpec((_ROWS_B, N_CLASSES), lambda i: (i, 0)),
        ],
        out_specs=pl.BlockSpec((_ROWS_B, N_CLASSES), lambda i: (i, 0)),
        out_shape=jax.ShapeDtypeStruct((N_PAD, N_CLASSES), jnp.float32),
    )(P2, cnt, q)
    return out[:N_NODES]


# speedup vs baseline: 2.8420x; 2.8420x over previous
import functools

import jax
import jax.numpy as jnp
from jax import lax
from jax.experimental import pallas as pl
from jax.experimental.pallas import tpu as pltpu
from jax.experimental.pallas import tpu_sc as plsc

N_NODES = 10000
N_EDGES = 320000
D_IN = 128
D_HID = 256
N_CLASSES = 64

NC = 2
NS = 16
N_PAD = 10240
BLK = 128
SUB = 8
EDGE_ROWS = 2500
EDGE_ROWS_PAD = 2560
BLKS_PER_TILE = EDGE_ROWS_PAD // (NC * NS)
NCHUNK = BLKS_PER_TILE // SUB
NODES_PER_TILE = N_PAD // NS


def _seg_sum_body(d, with_count, x_hbm, src_hbm, dst_hbm, *refs):
    if with_count:
        (out_hbm, cnt_hbm, src_idx, dst_idx, rows, ones, zcnt,
         acc, cnt_acc, gsem, csem) = refs
    else:
        (out_hbm, src_idx, dst_idx, rows, acc, gsem) = refs
        cnt_hbm = cnt_acc = ones = zcnt = csem = None

    cid = lax.axis_index("c")
    sid = lax.axis_index("s")
    wid = cid * NS + sid

    def zero_rows(i, _):
        for k in range(d // 16):
            rows[i, pl.ds(k * 16, 16)] = jnp.zeros((16,), jnp.float32)
        return _
    lax.fori_loop(0, BLK, zero_rows, None)
    if with_count:
        def fill_cnt(i, _):
            ones[pl.ds(i * 16, 16)] = jnp.full((16,), 1.0, jnp.float32)
            return _
        lax.fori_loop(0, BLK // 16, fill_cnt, None)
        def zero_zcnt(i, _):
            zcnt[pl.ds(i * 16, 16)] = jnp.zeros((16,), jnp.float32)
            return _
        lax.fori_loop(0, NODES_PER_TILE // 16, zero_zcnt, None)

    base_n = sid * NODES_PER_TILE
    for k in range(NODES_PER_TILE // BLK):
        pltpu.sync_copy(rows, acc.at[pl.ds(base_n + k * BLK, BLK)])
    if with_count:
        pltpu.sync_copy(zcnt, cnt_acc.at[pl.ds(base_n, NODES_PER_TILE)])
    plsc.subcore_barrier()

    base_c = wid * NCHUNK
    pltpu.sync_copy(src_hbm.at[pl.ds(base_c, NCHUNK)], src_idx)
    pltpu.sync_copy(dst_hbm.at[pl.ds(base_c, NCHUNK)], dst_idx)

    nblk = jnp.where(wid == NC * NS - 1,
                     BLKS_PER_TILE - (EDGE_ROWS_PAD - EDGE_ROWS),
                     BLKS_PER_TILE)

    def edge_block(j, _):
        r = j // SUB
        s = j % SUB
        pltpu.async_copy(x_hbm.at[src_idx.at[r, s]], rows, gsem).wait()
        pltpu.sync_copy(rows, acc.at[dst_idx.at[r, s]], add=True)
        if with_count:
            pltpu.async_copy(ones, cnt_acc.at[dst_idx.at[r, s]],
                             csem, add=True)
        return _
    lax.fori_loop(0, nblk, edge_block, None)
    if with_count:
        def drain_cnt(j, _):
            pltpu.make_async_copy(ones, cnt_acc.at[dst_idx.at[0, 0]],
                                  csem).wait()
            return _
        lax.fori_loop(0, nblk, drain_cnt, None)
    plsc.subcore_barrier()

    pltpu.sync_copy(acc.at[pl.ds(base_n, NODES_PER_TILE)],
                    out_hbm.at[cid, pl.ds(base_n, NODES_PER_TILE)])
    if with_count:
        pltpu.sync_copy(cnt_acc.at[pl.ds(base_n, NODES_PER_TILE)],
                        cnt_hbm.at[cid, pl.ds(base_n, NODES_PER_TILE)])


def _make_seg_sum(d, with_count):
    mesh = plsc.VectorSubcoreMesh(core_axis_name="c", subcore_axis_name="s",
                                  num_cores=NC, num_subcores=NS)
    out_type = [jax.ShapeDtypeStruct((NC, N_PAD, d), jnp.float32)]
    scratch = [
        pltpu.VMEM((NCHUNK, SUB, BLK), jnp.int32),
        pltpu.VMEM((NCHUNK, SUB, BLK), jnp.int32),
        pltpu.VMEM((BLK, d), jnp.float32),
    ]
    if with_count:
        out_type.append(jax.ShapeDtypeStruct((NC, N_PAD), jnp.float32))
        scratch += [
            pltpu.VMEM((BLK,), jnp.float32),
            pltpu.VMEM((NODES_PER_TILE,), jnp.float32),
        ]
    scratch += [pltpu.VMEM_SHARED((N_PAD, d), jnp.float32)]
    if with_count:
        scratch += [pltpu.VMEM_SHARED((N_PAD,), jnp.float32)]
    scratch += [pltpu.SemaphoreType.DMA] * (2 if with_count else 1)
    return pl.kernel(functools.partial(_seg_sum_body, d, with_count),
                     out_type=out_type, mesh=mesh, scratch_types=scratch,
                     name=f"sage_seg_sum_d{d}")


def _layer1_tc(P_ref, cnt_ref, x_ref, Wl1_ref, bl1_ref, Wr1_ref,
               Wl2_ref, bl2_ref, Wr2_ref, p_ref, q_ref):
    c = (cnt_ref[0] + cnt_ref[1]).reshape(-1, 1)
    mean = (P_ref[0] + P_ref[1]) * (1.0 / jnp.maximum(c, 1.0))
    h = jnp.dot(mean, Wl1_ref[...], preferred_element_type=jnp.float32)
    h = h + jnp.dot(x_ref[...], Wr1_ref[...], preferred_element_type=jnp.float32)
    h = jnp.maximum(h + bl1_ref[...], 0.0)
    p = jnp.dot(h, Wl2_ref[...], preferred_element_type=jnp.float32)
    p_ref[...] = jnp.concatenate(
        [p, jnp.zeros_like(p)], axis=1)
    q_ref[...] = (jnp.dot(h, Wr2_ref[...], preferred_element_type=jnp.float32)
                  + bl2_ref[...])


def _layer2_tc(P2_ref, cnt_ref, q_ref, o_ref):
    c = (cnt_ref[0] + cnt_ref[1]).reshape(-1, 1)
    agg = (P2_ref[0] + P2_ref[1])[:, :N_CLASSES]
    z = agg * (1.0 / jnp.maximum(c, 1.0)) + q_ref[...]
    m = jnp.max(z, axis=1, keepdims=True)
    e = jnp.exp(z - m)
    s = jnp.sum(e, axis=1, keepdims=True)
    o_ref[...] = z - m - jnp.log(s)


_ROWS_B = 1024


def kernel(x, edge_index, batch, Wl1, bl1, Wr1, Wl2, bl2, Wr2):
    del batch
    src = edge_index[0].astype(jnp.int32)
    dst = edge_index[1].astype(jnp.int32)
    pad = EDGE_ROWS_PAD * BLK - N_EDGES
    trash = N_NODES + jnp.arange(pad, dtype=jnp.int32) % (N_PAD - N_NODES)
    src3d = jnp.pad(src, (0, pad)).reshape(EDGE_ROWS_PAD // SUB, SUB, BLK)
    dst3d = jnp.concatenate([dst, trash]).reshape(
        EDGE_ROWS_PAD // SUB, SUB, BLK)
    xp = jnp.pad(x, ((0, N_PAD - N_NODES), (0, 0)))

    P1, cnt = _make_seg_sum(D_IN, True)(xp, src3d, dst3d)

    grid = (N_PAD // _ROWS_B,)
    p, q = pl.pallas_call(
        _layer1_tc,
        grid=grid,
        in_specs=[
            pl.BlockSpec((NC, _ROWS_B, D_IN), lambda i: (0, i, 0)),
            pl.BlockSpec((NC, _ROWS_B), lambda i: (0, i)),
            pl.BlockSpec((_ROWS_B, D_IN), lambda i: (i, 0)),
            pl.BlockSpec((D_IN, D_HID), lambda i: (0, 0)),
            pl.BlockSpec((1, D_HID), lambda i: (0, 0)),
            pl.BlockSpec((D_IN, D_HID), lambda i: (0, 0)),
            pl.BlockSpec((D_HID, N_CLASSES), lambda i: (0, 0)),
            pl.BlockSpec((1, N_CLASSES), lambda i: (0, 0)),
            pl.BlockSpec((D_HID, N_CLASSES), lambda i: (0, 0)),
        ],
        out_specs=[
            pl.BlockSpec((_ROWS_B, 2 * N_CLASSES), lambda i: (i, 0)),
            pl.BlockSpec((_ROWS_B, N_CLASSES), lambda i: (i, 0)),
        ],
        out_shape=[
            jax.ShapeDtypeStruct((N_PAD, 2 * N_CLASSES), jnp.float32),
            jax.ShapeDtypeStruct((N_PAD, N_CLASSES), jnp.float32),
        ],
    )(P1, cnt, xp, Wl1, bl1.reshape(1, D_HID), Wr1,
      Wl2, bl2.reshape(1, N_CLASSES), Wr2)

    (P2,) = _make_seg_sum(2 * N_CLASSES, False)(p, src3d, dst3d)

    out = pl.pallas_call(
        _layer2_tc,
        grid=grid,
        in_specs=[
            pl.BlockSpec((NC, _ROWS_B, 2 * N_CLASSES), lambda i: (0, i, 0)),
            pl.BlockSpec((NC, _ROWS_B), lambda i: (0, i)),
            pl.BlockSpec((_ROWS_B, N_CLASSES), lambda i: (i, 0)),
        ],
        out_specs=pl.BlockSpec((_ROWS_B, N_CLASSES), lambda i: (i, 0)),
        out_shape=jax.ShapeDtypeStruct((N_PAD, N_CLASSES), jnp.float32),
    )(P2, cnt, q)
    return out[:N_NODES]


# speedup vs baseline: 4.1384x; 1.4562x over previous
import functools

import jax
import jax.numpy as jnp
from jax import lax
from jax.experimental import pallas as pl
from jax.experimental.pallas import tpu as pltpu
from jax.experimental.pallas import tpu_sc as plsc

N_NODES = 10000
N_EDGES = 320000
D_IN = 128
D_HID = 256
N_CLASSES = 64

NC = 2
NS = 16
N_PAD = 10240
BLK = 128
SUB = 8
EDGE_ROWS = 2500
EDGE_ROWS_PAD = 2560
BLKS_PER_TILE = EDGE_ROWS_PAD // (NC * NS)
NCHUNK = BLKS_PER_TILE // SUB
HALF_CHUNKS = NCHUNK // 2
NODES_PER_TILE = N_PAD // NS


def _seg_sum_body(d, with_count, x_hbm, src_hbm, dst_hbm, *refs):
    if with_count:
        (out_hbm, cnt_hbm, src_idx, dst_idx, rows, rows1, ones,
         acc, cnt_acc, gsem, gsem1, csem) = refs
    else:
        (out_hbm, src_idx, dst_idx, rows, rows1, acc, gsem, gsem1) = refs
        cnt_hbm = cnt_acc = ones = csem = None

    cid = lax.axis_index("c")
    sid = lax.axis_index("s")
    wid = cid * NS + sid

    def zero_rows(i, _):
        for k in range(d // 16):
            rows[i, pl.ds(k * 16, 16)] = jnp.zeros((16,), jnp.float32)
        return _
    lax.fori_loop(0, BLK, zero_rows, None)
    if with_count:
        def fill_cnt(i, _):
            ones[pl.ds(i * 16, 16)] = jnp.full((16,), 1.0, jnp.float32)
            return _
        lax.fori_loop(0, BLK // 16, fill_cnt, None)

    base_n = sid * NODES_PER_TILE
    for k in range(NODES_PER_TILE // BLK):
        pltpu.sync_copy(rows, acc.at[pl.ds(base_n + k * BLK, BLK)])
        if with_count:
            pltpu.sync_copy(rows.at[0],
                            cnt_acc.at[pl.ds(base_n + k * BLK, BLK)])
    plsc.subcore_barrier()

    base_c = wid * NCHUNK
    nblk = jnp.where(wid == NC * NS - 1,
                     BLKS_PER_TILE - (EDGE_ROWS_PAD - EDGE_ROWS),
                     BLKS_PER_TILE)
    HALF = HALF_CHUNKS * SUB

    def fire(j, buf, sem):
        pltpu.async_copy(x_hbm.at[src_idx.at[j // SUB, j % SUB]], buf, sem)

    def wait(buf, sem):
        pltpu.make_async_copy(x_hbm.at[src_idx.at[0, 0]], buf, sem).wait()

    def scat(j, buf):
        pltpu.sync_copy(buf, acc.at[dst_idx.at[j // SUB, j % SUB]], add=True)
        if with_count:
            pltpu.async_copy(ones, cnt_acc.at[dst_idx.at[j // SUB, j % SUB]],
                             csem, add=True)

    for h in range(BLKS_PER_TILE // HALF):
        pltpu.sync_copy(src_hbm.at[pl.ds(base_c + h * HALF_CHUNKS,
                                         HALF_CHUNKS)], src_idx)
        pltpu.sync_copy(dst_hbm.at[pl.ds(base_c + h * HALF_CHUNKS,
                                         HALF_CHUNKS)], dst_idx)
        t = jnp.clip(nblk - h * HALF, 0, HALF)

        @pl.when(t > 0)
        def _prime():
            fire(0, rows, gsem)

        def pair(j2, _):
            j0 = 2 * j2
            j1 = j0 + 1
            fire(j1, rows1, gsem1)
            wait(rows, gsem)
            scat(j0, rows)

            @pl.when(j1 + 1 < t)
            def _next():
                fire(j1 + 1, rows, gsem)
            wait(rows1, gsem1)
            scat(j1, rows1)
            return _
        lax.fori_loop(0, t // 2, pair, None)
        if with_count:
            def drain_cnt(j, _):
                pltpu.make_async_copy(ones, cnt_acc.at[dst_idx.at[0, 0]],
                                      csem).wait()
                return _
            lax.fori_loop(0, t, drain_cnt, None)
    plsc.subcore_barrier()

    pltpu.sync_copy(acc.at[pl.ds(base_n, NODES_PER_TILE)],
                    out_hbm.at[cid, pl.ds(base_n, NODES_PER_TILE)])
    if with_count:
        pltpu.sync_copy(cnt_acc.at[pl.ds(base_n, NODES_PER_TILE)],
                        cnt_hbm.at[cid, pl.ds(base_n, NODES_PER_TILE)])


def _make_seg_sum(d, with_count):
    mesh = plsc.VectorSubcoreMesh(core_axis_name="c", subcore_axis_name="s",
                                  num_cores=NC, num_subcores=NS)
    out_type = [jax.ShapeDtypeStruct((NC, N_PAD, d), jnp.float32)]
    scratch = [
        pltpu.VMEM((HALF_CHUNKS, SUB, BLK), jnp.int32),
        pltpu.VMEM((HALF_CHUNKS, SUB, BLK), jnp.int32),
        pltpu.VMEM((BLK, d), jnp.float32),
        pltpu.VMEM((BLK, d), jnp.float32),
    ]
    if with_count:
        out_type.append(jax.ShapeDtypeStruct((NC, N_PAD), jnp.float32))
        scratch += [pltpu.VMEM((BLK,), jnp.float32)]
    scratch += [pltpu.VMEM_SHARED((N_PAD, d), jnp.float32)]
    if with_count:
        scratch += [pltpu.VMEM_SHARED((N_PAD,), jnp.float32)]
    scratch += [pltpu.SemaphoreType.DMA] * (3 if with_count else 2)
    return pl.kernel(functools.partial(_seg_sum_body, d, with_count),
                     out_type=out_type, mesh=mesh, scratch_types=scratch,
                     name=f"sage_seg_sum_d{d}")


def _layer1_tc(P_ref, cnt_ref, x_ref, Wl1_ref, bl1_ref, Wr1_ref,
               Wl2_ref, bl2_ref, Wr2_ref, p_ref, q_ref):
    c = cnt_ref[0] + cnt_ref[1]
    mean = (P_ref[0] + P_ref[1]) * (1.0 / jnp.maximum(c, 1.0))
    h = jnp.dot(mean, Wl1_ref[...], preferred_element_type=jnp.float32)
    h = h + jnp.dot(x_ref[...], Wr1_ref[...], preferred_element_type=jnp.float32)
    h = jnp.maximum(h + bl1_ref[...], 0.0)
    p = jnp.dot(h, Wl2_ref[...], preferred_element_type=jnp.float32)
    p_ref[...] = jnp.concatenate(
        [p, jnp.zeros_like(p)], axis=1)
    q_ref[...] = (jnp.dot(h, Wr2_ref[...], preferred_element_type=jnp.float32)
                  + bl2_ref[...])


def _layer2_tc(P2_ref, cnt_ref, q_ref, o_ref):
    c = cnt_ref[0] + cnt_ref[1]
    agg = (P2_ref[0] + P2_ref[1])[:, :N_CLASSES]
    z = agg * (1.0 / jnp.maximum(c, 1.0)) + q_ref[...]
    m = jnp.max(z, axis=1, keepdims=True)
    e = jnp.exp(z - m)
    s = jnp.sum(e, axis=1, keepdims=True)
    o_ref[...] = z - m - jnp.log(s)


_ROWS_B = 2000


def kernel(x, edge_index, batch, Wl1, bl1, Wr1, Wl2, bl2, Wr2):
    del batch
    src = edge_index[0].astype(jnp.int32)
    dst = edge_index[1].astype(jnp.int32)
    pad = EDGE_ROWS_PAD * BLK - N_EDGES
    trash = N_NODES + jnp.arange(pad, dtype=jnp.int32) % (N_PAD - N_NODES)
    src3d = jnp.pad(src, (0, pad)).reshape(EDGE_ROWS_PAD // SUB, SUB, BLK)
    dst3d = jnp.concatenate([dst, trash]).reshape(
        EDGE_ROWS_PAD // SUB, SUB, BLK)

    P1, cnt = _make_seg_sum(D_IN, True)(x, src3d, dst3d)
    cnt3 = cnt.reshape(NC, N_PAD, 1)

    grid = (N_NODES // _ROWS_B,)
    p, q = pl.pallas_call(
        _layer1_tc,
        grid=grid,
        in_specs=[
            pl.BlockSpec((NC, _ROWS_B, D_IN), lambda i: (0, i, 0)),
            pl.BlockSpec((NC, _ROWS_B, 1), lambda i: (0, i, 0)),
            pl.BlockSpec((_ROWS_B, D_IN), lambda i: (i, 0)),
            pl.BlockSpec((D_IN, D_HID), lambda i: (0, 0)),
            pl.BlockSpec((1, D_HID), lambda i: (0, 0)),
            pl.BlockSpec((D_IN, D_HID), lambda i: (0, 0)),
            pl.BlockSpec((D_HID, N_CLASSES), lambda i: (0, 0)),
            pl.BlockSpec((1, N_CLASSES), lambda i: (0, 0)),
            pl.BlockSpec((D_HID, N_CLASSES), lambda i: (0, 0)),
        ],
        out_specs=[
            pl.BlockSpec((_ROWS_B, 2 * N_CLASSES), lambda i: (i, 0)),
            pl.BlockSpec((_ROWS_B, N_CLASSES), lambda i: (i, 0)),
        ],
        out_shape=[
            jax.ShapeDtypeStruct((N_NODES, 2 * N_CLASSES), jnp.float32),
            jax.ShapeDtypeStruct((N_NODES, N_CLASSES), jnp.float32),
        ],
    )(P1, cnt3, x, Wl1, bl1.reshape(1, D_HID), Wr1,
      Wl2, bl2.reshape(1, N_CLASSES), Wr2)

    (P2,) = _make_seg_sum(2 * N_CLASSES, False)(p, src3d, dst3d)

    out = pl.pallas_call(
        _layer2_tc,
        grid=grid,
        in_specs=[
            pl.BlockSpec((NC, _ROWS_B, 2 * N_CLASSES), lambda i: (0, i, 0)),
            pl.BlockSpec((NC, _ROWS_B, 1), lambda i: (0, i, 0)),
            pl.BlockSpec((_ROWS_B, N_CLASSES), lambda i: (i, 0)),
        ],
        out_specs=pl.BlockSpec((_ROWS_B, N_CLASSES), lambda i: (i, 0)),
        out_shape=jax.ShapeDtypeStruct((N_NODES, N_CLASSES), jnp.float32),
    )(P2, cnt3, q)
    return out


# speedup vs baseline: 4.7377x; 1.1448x over previous
import functools

import jax
import jax.numpy as jnp
from jax import lax
from jax.experimental import pallas as pl
from jax.experimental.pallas import tpu as pltpu
from jax.experimental.pallas import tpu_sc as plsc

N_NODES = 10000
N_EDGES = 320000
D_IN = 128
D_HID = 256
N_CLASSES = 64

NC = 2
NS = 16
N_PAD = 10240
BLK = 128
SUB = 8
EDGE_ROWS = 2500
EDGE_ROWS_PAD = 2560
BLKS_PER_TILE = EDGE_ROWS_PAD // (NC * NS)
NCHUNK = BLKS_PER_TILE // SUB
HALF_CHUNKS = NCHUNK // 2
NODES_PER_TILE = N_PAD // NS


def _seg_sum_body(d, with_count, x_hbm, src_hbm, dst_hbm, *refs):
    if with_count:
        (out_hbm, cnt_hbm, src_idx, dst_idx, rows, rows1, ones,
         acc, cnt_acc, gsem, gsem1, isem, csem) = refs
    else:
        (out_hbm, src_idx, dst_idx, rows, rows1, acc,
         gsem, gsem1, isem) = refs
        cnt_hbm = cnt_acc = ones = csem = None

    cid = lax.axis_index("c")
    sid = lax.axis_index("s")
    wid = cid * NS + sid
    base_c = wid * NCHUNK

    pltpu.async_copy(src_hbm.at[pl.ds(base_c, HALF_CHUNKS)], src_idx, isem)
    pltpu.async_copy(dst_hbm.at[pl.ds(base_c, HALF_CHUNKS)], dst_idx, isem)

    def zero_rows(i, _):
        for k in range(d // 16):
            rows[i, pl.ds(k * 16, 16)] = jnp.zeros((16,), jnp.float32)
        return _
    lax.fori_loop(0, BLK, zero_rows, None)
    if with_count:
        def fill_cnt(i, _):
            ones[pl.ds(i * 16, 16)] = jnp.full((16,), 1.0, jnp.float32)
            return _
        lax.fori_loop(0, BLK // 16, fill_cnt, None)

    base_n = sid * NODES_PER_TILE
    for k in range(NODES_PER_TILE // BLK):
        pltpu.sync_copy(rows, acc.at[pl.ds(base_n + k * BLK, BLK)])
        if with_count:
            pltpu.sync_copy(rows.at[0],
                            cnt_acc.at[pl.ds(base_n + k * BLK, BLK)])
    plsc.subcore_barrier()

    nblk = jnp.where(wid == NC * NS - 1,
                     BLKS_PER_TILE - (EDGE_ROWS_PAD - EDGE_ROWS),
                     BLKS_PER_TILE)
    HALF = HALF_CHUNKS * SUB

    def fire(j, buf, sem):
        pltpu.async_copy(x_hbm.at[src_idx.at[j // SUB, j % SUB]], buf, sem)

    def wait(buf, sem):
        pltpu.make_async_copy(x_hbm.at[src_idx.at[0, 0]], buf, sem).wait()

    def scat(j, buf):
        pltpu.sync_copy(buf, acc.at[dst_idx.at[j // SUB, j % SUB]], add=True)
        if with_count:
            pltpu.async_copy(ones, cnt_acc.at[dst_idx.at[j // SUB, j % SUB]],
                             csem, add=True)

    for h in range(BLKS_PER_TILE // HALF):
        if h == 0:
            pltpu.make_async_copy(src_hbm.at[pl.ds(base_c, HALF_CHUNKS)],
                                  src_idx, isem).wait()
            pltpu.make_async_copy(dst_hbm.at[pl.ds(base_c, HALF_CHUNKS)],
                                  dst_idx, isem).wait()
        else:
            pltpu.sync_copy(src_hbm.at[pl.ds(base_c + h * HALF_CHUNKS,
                                             HALF_CHUNKS)], src_idx)
            pltpu.sync_copy(dst_hbm.at[pl.ds(base_c + h * HALF_CHUNKS,
                                             HALF_CHUNKS)], dst_idx)
        t = jnp.clip(nblk - h * HALF, 0, HALF)

        @pl.when(t > 0)
        def _prime():
            fire(0, rows, gsem)

        def pair(j2, _):
            j0 = 2 * j2
            j1 = j0 + 1
            fire(j1, rows1, gsem1)
            wait(rows, gsem)
            scat(j0, rows)

            @pl.when(j1 + 1 < t)
            def _next():
                fire(j1 + 1, rows, gsem)
            wait(rows1, gsem1)
            scat(j1, rows1)
            return _
        lax.fori_loop(0, t // 2, pair, None)
        if with_count:
            def drain_cnt(j, _):
                pltpu.make_async_copy(ones, cnt_acc.at[dst_idx.at[0, 0]],
                                      csem).wait()
                return _
            lax.fori_loop(0, t, drain_cnt, None)
    plsc.subcore_barrier()

    pltpu.sync_copy(acc.at[pl.ds(base_n, NODES_PER_TILE)],
                    out_hbm.at[cid, pl.ds(base_n, NODES_PER_TILE)])
    if with_count:
        pltpu.sync_copy(cnt_acc.at[pl.ds(base_n, NODES_PER_TILE)],
                        cnt_hbm.at[cid, pl.ds(base_n, NODES_PER_TILE)])


def _make_seg_sum(d, with_count):
    mesh = plsc.VectorSubcoreMesh(core_axis_name="c", subcore_axis_name="s",
                                  num_cores=NC, num_subcores=NS)
    out_type = [jax.ShapeDtypeStruct((NC, N_PAD, d), jnp.float32)]
    scratch = [
        pltpu.VMEM((HALF_CHUNKS, SUB, BLK), jnp.int32),
        pltpu.VMEM((HALF_CHUNKS, SUB, BLK), jnp.int32),
        pltpu.VMEM((BLK, d), jnp.float32),
        pltpu.VMEM((BLK, d), jnp.float32),
    ]
    if with_count:
        out_type.append(jax.ShapeDtypeStruct((NC, N_PAD), jnp.float32))
        scratch += [pltpu.VMEM((BLK,), jnp.float32)]
    scratch += [pltpu.VMEM_SHARED((N_PAD, d), jnp.float32)]
    if with_count:
        scratch += [pltpu.VMEM_SHARED((N_PAD,), jnp.float32)]
    scratch += [pltpu.SemaphoreType.DMA] * (4 if with_count else 3)
    return pl.kernel(functools.partial(_seg_sum_body, d, with_count),
                     out_type=out_type, mesh=mesh, scratch_types=scratch,
                     name=f"sage_seg_sum_d{d}")


def _layer1_tc(P_ref, cnt_ref, x_ref, Wl1_ref, bl1_ref, Wr1_ref,
               Wl2_ref, bl2_ref, Wr2_ref, p_ref, q_ref):
    c = cnt_ref[0] + cnt_ref[1]
    mean = (P_ref[0] + P_ref[1]) * (1.0 / jnp.maximum(c, 1.0))
    h = jnp.dot(mean, Wl1_ref[...], preferred_element_type=jnp.float32)
    h = h + jnp.dot(x_ref[...], Wr1_ref[...], preferred_element_type=jnp.float32)
    h = jnp.maximum(h + bl1_ref[...], 0.0)
    p = jnp.dot(h, Wl2_ref[...], preferred_element_type=jnp.float32)
    p_ref[...] = jnp.concatenate(
        [p, jnp.zeros_like(p)], axis=1)
    q_ref[...] = (jnp.dot(h, Wr2_ref[...], preferred_element_type=jnp.float32)
                  + bl2_ref[...])


def _layer2_tc(P2_ref, cnt_ref, q_ref, o_ref):
    c = cnt_ref[0] + cnt_ref[1]
    agg = (P2_ref[0] + P2_ref[1])[:, :N_CLASSES]
    z = agg * (1.0 / jnp.maximum(c, 1.0)) + q_ref[...]
    m = jnp.max(z, axis=1, keepdims=True)
    e = jnp.exp(z - m)
    s = jnp.sum(e, axis=1, keepdims=True)
    o_ref[...] = z - m - jnp.log(s)


_ROWS_B = 2000


def kernel(x, edge_index, batch, Wl1, bl1, Wr1, Wl2, bl2, Wr2):
    del batch
    pad = EDGE_ROWS_PAD * BLK - N_EDGES
    ep = jnp.pad(edge_index.astype(jnp.int32), ((0, 0), (0, pad)))
    src3d = ep[0].reshape(EDGE_ROWS_PAD // SUB, SUB, BLK)
    dst3d = ep[1].reshape(EDGE_ROWS_PAD // SUB, SUB, BLK)

    P1, cnt = _make_seg_sum(D_IN, True)(x, src3d, dst3d)
    cnt3 = cnt.reshape(NC, N_PAD, 1)

    grid = (N_NODES // _ROWS_B,)
    p, q = pl.pallas_call(
        _layer1_tc,
        grid=grid,
        in_specs=[
            pl.BlockSpec((NC, _ROWS_B, D_IN), lambda i: (0, i, 0)),
            pl.BlockSpec((NC, _ROWS_B, 1), lambda i: (0, i, 0)),
            pl.BlockSpec((_ROWS_B, D_IN), lambda i: (i, 0)),
            pl.BlockSpec((D_IN, D_HID), lambda i: (0, 0)),
            pl.BlockSpec((1, D_HID), lambda i: (0, 0)),
            pl.BlockSpec((D_IN, D_HID), lambda i: (0, 0)),
            pl.BlockSpec((D_HID, N_CLASSES), lambda i: (0, 0)),
            pl.BlockSpec((1, N_CLASSES), lambda i: (0, 0)),
            pl.BlockSpec((D_HID, N_CLASSES), lambda i: (0, 0)),
        ],
        out_specs=[
            pl.BlockSpec((_ROWS_B, 2 * N_CLASSES), lambda i: (i, 0)),
            pl.BlockSpec((_ROWS_B, N_CLASSES), lambda i: (i, 0)),
        ],
        out_shape=[
            jax.ShapeDtypeStruct((N_NODES, 2 * N_CLASSES), jnp.float32),
            jax.ShapeDtypeStruct((N_NODES, N_CLASSES), jnp.float32),
        ],
    )(P1, cnt3, x, Wl1, bl1.reshape(1, D_HID), Wr1,
      Wl2, bl2.reshape(1, N_CLASSES), Wr2)

    (P2,) = _make_seg_sum(2 * N_CLASSES, False)(p, src3d, dst3d)

    out = pl.pallas_call(
        _layer2_tc,
        grid=grid,
        in_specs=[
            pl.BlockSpec((NC, _ROWS_B, 2 * N_CLASSES), lambda i: (0, i, 0)),
            pl.BlockSpec((NC, _ROWS_B, 1), lambda i: (0, i, 0)),
            pl.BlockSpec((_ROWS_B, N_CLASSES), lambda i: (i, 0)),
        ],
        out_specs=pl.BlockSpec((_ROWS_B, N_CLASSES), lambda i: (i, 0)),
        out_shape=jax.ShapeDtypeStruct((N_NODES, N_CLASSES), jnp.float32),
    )(P2, cnt3, q)
    return out


# speedup vs baseline: 4.7454x; 1.0016x over previous
import functools

import jax
import jax.numpy as jnp
from jax import lax
from jax.experimental import pallas as pl
from jax.experimental.pallas import tpu as pltpu
from jax.experimental.pallas import tpu_sc as plsc

N_NODES = 10000
N_EDGES = 320000
D_IN = 128
D_HID = 256
N_CLASSES = 64

NC = 2
NS = 16
N_PAD = 10240
BLK = 128
SUB = 8
EDGE_ROWS = 2500
EDGE_ROWS_PAD = 2560
BLKS_PER_TILE = EDGE_ROWS_PAD // (NC * NS)
NCHUNK = BLKS_PER_TILE // SUB
HALF_CHUNKS = NCHUNK // 2
NODES_PER_TILE = N_PAD // NS


def _seg_sum_body(d, with_count, x_hbm, src_hbm, dst_hbm, *refs):
    if with_count:
        (out_hbm, cnt_hbm, src_idx, dst_idx, rows, rows1, ones,
         acc, cnt_acc, gsem, gsem1, isem, csem) = refs
    else:
        (out_hbm, src_idx, dst_idx, rows, rows1, acc,
         gsem, gsem1, isem) = refs
        cnt_hbm = cnt_acc = ones = csem = None

    cid = lax.axis_index("c")
    sid = lax.axis_index("s")
    wid = cid * NS + sid
    base_c = wid * NCHUNK

    pltpu.async_copy(src_hbm.at[pl.ds(base_c, HALF_CHUNKS)], src_idx, isem)
    pltpu.async_copy(dst_hbm.at[pl.ds(base_c, HALF_CHUNKS)], dst_idx, isem)

    def zero_rows(i, _):
        for k in range(d // 16):
            rows[i, pl.ds(k * 16, 16)] = jnp.zeros((16,), jnp.float32)
        return _
    lax.fori_loop(0, BLK, zero_rows, None)
    if with_count:
        def fill_cnt(i, _):
            ones[pl.ds(i * 16, 16)] = jnp.full((16,), 1.0, jnp.float32)
            return _
        lax.fori_loop(0, BLK // 16, fill_cnt, None)

    base_n = sid * NODES_PER_TILE
    for k in range(NODES_PER_TILE // BLK):
        pltpu.sync_copy(rows, acc.at[pl.ds(base_n + k * BLK, BLK)])
        if with_count:
            pltpu.sync_copy(rows.at[0],
                            cnt_acc.at[pl.ds(base_n + k * BLK, BLK)])
    plsc.subcore_barrier()

    nblk = jnp.where(wid == NC * NS - 1,
                     BLKS_PER_TILE - (EDGE_ROWS_PAD - EDGE_ROWS),
                     BLKS_PER_TILE)
    HALF = HALF_CHUNKS * SUB

    def fire(j, buf, sem):
        pltpu.async_copy(x_hbm.at[src_idx.at[j // SUB, j % SUB]], buf, sem)

    def wait(buf, sem):
        pltpu.make_async_copy(x_hbm.at[src_idx.at[0, 0]], buf, sem).wait()

    def scat(j, buf):
        pltpu.sync_copy(buf, acc.at[dst_idx.at[j // SUB, j % SUB]], add=True)
        if with_count:
            pltpu.async_copy(ones, cnt_acc.at[dst_idx.at[j // SUB, j % SUB]],
                             csem, add=True)

    for h in range(BLKS_PER_TILE // HALF):
        if h == 0:
            pltpu.make_async_copy(src_hbm.at[pl.ds(base_c, HALF_CHUNKS)],
                                  src_idx, isem).wait()
            pltpu.make_async_copy(dst_hbm.at[pl.ds(base_c, HALF_CHUNKS)],
                                  dst_idx, isem).wait()
        else:
            pltpu.sync_copy(src_hbm.at[pl.ds(base_c + h * HALF_CHUNKS,
                                             HALF_CHUNKS)], src_idx)
            pltpu.sync_copy(dst_hbm.at[pl.ds(base_c + h * HALF_CHUNKS,
                                             HALF_CHUNKS)], dst_idx)
        t = jnp.clip(nblk - h * HALF, 0, HALF)

        @pl.when(t > 0)
        def _prime():
            fire(0, rows, gsem)

        def pair(j2, _):
            j0 = 2 * j2
            j1 = j0 + 1
            fire(j1, rows1, gsem1)
            wait(rows, gsem)
            scat(j0, rows)

            @pl.when(j1 + 1 < t)
            def _next():
                fire(j1 + 1, rows, gsem)
            wait(rows1, gsem1)
            scat(j1, rows1)
            return _
        lax.fori_loop(0, t // 2, pair, None)
        if with_count:
            def drain_cnt(j, _):
                pltpu.make_async_copy(ones, cnt_acc.at[dst_idx.at[0, 0]],
                                      csem).wait()
                return _
            lax.fori_loop(0, t, drain_cnt, None)
    plsc.subcore_barrier()

    pltpu.sync_copy(acc.at[pl.ds(base_n, NODES_PER_TILE)],
                    out_hbm.at[cid, pl.ds(base_n, NODES_PER_TILE)])
    if with_count:
        pltpu.sync_copy(cnt_acc.at[pl.ds(base_n, NODES_PER_TILE)],
                        cnt_hbm.at[cid, pl.ds(base_n, NODES_PER_TILE)])


def _make_seg_sum(d, with_count):
    mesh = plsc.VectorSubcoreMesh(core_axis_name="c", subcore_axis_name="s",
                                  num_cores=NC, num_subcores=NS)
    out_type = [jax.ShapeDtypeStruct((NC, N_PAD, d), jnp.float32)]
    scratch = [
        pltpu.VMEM((HALF_CHUNKS, SUB, BLK), jnp.int32),
        pltpu.VMEM((HALF_CHUNKS, SUB, BLK), jnp.int32),
        pltpu.VMEM((BLK, d), jnp.float32),
        pltpu.VMEM((BLK, d), jnp.float32),
    ]
    if with_count:
        out_type.append(jax.ShapeDtypeStruct((NC, N_PAD), jnp.float32))
        scratch += [pltpu.VMEM((BLK,), jnp.float32)]
    scratch += [pltpu.VMEM_SHARED((N_PAD, d), jnp.float32)]
    if with_count:
        scratch += [pltpu.VMEM_SHARED((N_PAD,), jnp.float32)]
    scratch += [pltpu.SemaphoreType.DMA] * (4 if with_count else 3)
    cp = (None if with_count else
          pltpu.CompilerParams(use_tc_tiling_on_sc=False))
    return pl.kernel(functools.partial(_seg_sum_body, d, with_count),
                     out_type=out_type, mesh=mesh, scratch_types=scratch,
                     compiler_params=cp, name=f"sage_seg_sum_d{d}")


def _layer1_tc(P_ref, cnt_ref, x_ref, Wl1_ref, bl1_ref, Wr1_ref,
               Wl2_ref, bl2_ref, Wr2_ref, p_ref, q_ref):
    c = cnt_ref[0] + cnt_ref[1]
    mean = (P_ref[0] + P_ref[1]) * (1.0 / jnp.maximum(c, 1.0))
    h = jnp.dot(mean, Wl1_ref[...], preferred_element_type=jnp.float32)
    h = h + jnp.dot(x_ref[...], Wr1_ref[...], preferred_element_type=jnp.float32)
    h = jnp.maximum(h + bl1_ref[...], 0.0)
    p_ref[...] = jnp.dot(h, Wl2_ref[...], preferred_element_type=jnp.float32)
    q_ref[...] = (jnp.dot(h, Wr2_ref[...], preferred_element_type=jnp.float32)
                  + bl2_ref[...])


def _layer2_tc(P2_ref, cnt_ref, q_ref, o_ref):
    c = cnt_ref[0] + cnt_ref[1]
    agg = P2_ref[0] + P2_ref[1]
    z = agg * (1.0 / jnp.maximum(c, 1.0)) + q_ref[...]
    m = jnp.max(z, axis=1, keepdims=True)
    e = jnp.exp(z - m)
    s = jnp.sum(e, axis=1, keepdims=True)
    o_ref[...] = z - m - jnp.log(s)


_ROWS_B = 2000


def kernel(x, edge_index, batch, Wl1, bl1, Wr1, Wl2, bl2, Wr2):
    del batch
    pad = EDGE_ROWS_PAD * BLK - N_EDGES
    ep = jnp.pad(edge_index.astype(jnp.int32), ((0, 0), (0, pad)))
    src3d = ep[0].reshape(EDGE_ROWS_PAD // SUB, SUB, BLK)
    dst3d = ep[1].reshape(EDGE_ROWS_PAD // SUB, SUB, BLK)

    P1, cnt = _make_seg_sum(D_IN, True)(x, src3d, dst3d)
    cnt3 = cnt.reshape(NC, N_PAD, 1)

    grid = (N_NODES // _ROWS_B,)
    p, q = pl.pallas_call(
        _layer1_tc,
        grid=grid,
        in_specs=[
            pl.BlockSpec((NC, _ROWS_B, D_IN), lambda i: (0, i, 0)),
            pl.BlockSpec((NC, _ROWS_B, 1), lambda i: (0, i, 0)),
            pl.BlockSpec((_ROWS_B, D_IN), lambda i: (i, 0)),
            pl.BlockSpec((D_IN, D_HID), lambda i: (0, 0)),
            pl.BlockSpec((1, D_HID), lambda i: (0, 0)),
            pl.BlockSpec((D_IN, D_HID), lambda i: (0, 0)),
            pl.BlockSpec((D_HID, N_CLASSES), lambda i: (0, 0)),
            pl.BlockSpec((1, N_CLASSES), lambda i: (0, 0)),
            pl.BlockSpec((D_HID, N_CLASSES), lambda i: (0, 0)),
        ],
        out_specs=[
            pl.BlockSpec((_ROWS_B, N_CLASSES), lambda i: (i, 0)),
            pl.BlockSpec((_ROWS_B, N_CLASSES), lambda i: (i, 0)),
        ],
        out_shape=[
            jax.ShapeDtypeStruct((N_NODES, N_CLASSES), jnp.float32),
            jax.ShapeDtypeStruct((N_NODES, N_CLASSES), jnp.float32),
        ],
    )(P1, cnt3, x, Wl1, bl1.reshape(1, D_HID), Wr1,
      Wl2, bl2.reshape(1, N_CLASSES), Wr2)

    (P2,) = _make_seg_sum(N_CLASSES, False)(p, src3d, dst3d)

    out = pl.pallas_call(
        _layer2_tc,
        grid=grid,
        in_specs=[
            pl.BlockSpec((NC, _ROWS_B, N_CLASSES), lambda i: (0, i, 0)),
            pl.BlockSpec((NC, _ROWS_B, 1), lambda i: (0, i, 0)),
            pl.BlockSpec((_ROWS_B, N_CLASSES), lambda i: (i, 0)),
        ],
        out_specs=pl.BlockSpec((_ROWS_B, N_CLASSES), lambda i: (i, 0)),
        out_shape=jax.ShapeDtypeStruct((N_NODES, N_CLASSES), jnp.float32),
    )(P2, cnt3, q)
    return out
